# Initial kernel scaffold; baseline (speedup 1.0000x reference)
import jax
import jax.numpy as jnp
from jax.experimental import pallas as pl


def kernel(feats, node2edge_idx, edge_emb, edge_node_adj, id_emb, W_prep, W_edge_prep, W_e_self_0, W_e_neigh_0, W_n_self_0, W_n_neigh_0, W_e_self_1, W_e_neigh_1, W_n_self_1, W_n_neigh_1):
    raise NotImplementedError("write your pallas kernel here")


# speedup vs baseline: 4.8210x; 4.8210x over previous
import functools

import jax
import jax.numpy as jnp
from jax import lax
from jax.experimental import pallas as pl
from jax.experimental.pallas import tpu as pltpu
from jax.experimental.pallas import tpu_sc as plsc

NC = 2
NS = 16
NW = NC * NS
LANES = 16


def _sc_mesh():
  return plsc.VectorSubcoreMesh(
      core_axis_name="c", subcore_axis_name="s", num_cores=NC, num_subcores=NS)


_SC_PARAMS = pltpu.CompilerParams(use_tc_tiling_on_sc=False)


def _tc_node_prep(feats, W_prep, W_e_neigh_0):
  N, DF = feats.shape
  P = W_prep.shape[1]
  OUT = W_e_neigh_0.shape[1]
  BN = 2000

  def body(f_ref, wp_ref, wn_ref, o_ref):
    f0 = jnp.dot(f_ref[...], wp_ref[...], preferred_element_type=jnp.float32)
    o_ref[...] = 0.5 * jnp.dot(f0, wn_ref[...],
                               preferred_element_type=jnp.float32)

  return pl.pallas_call(
      body,
      grid=(N // BN,),
      in_specs=[
          pl.BlockSpec((BN, DF), lambda i: (i, 0)),
          pl.BlockSpec((DF, P), lambda i: (0, 0)),
          pl.BlockSpec((P, OUT), lambda i: (0, 0)),
      ],
      out_specs=pl.BlockSpec((BN, OUT), lambda i: (i, 0)),
      out_shape=jax.ShapeDtypeStruct((N, OUT), jnp.float32),
  )(feats, W_prep, W_e_neigh_0)


def _tc_edge_prep(edge_emb, W_edge_prep, W_e_self_0):
  E, DE = edge_emb.shape
  P = W_edge_prep.shape[1]
  OUT = W_e_self_0.shape[1]
  BE = 2000

  def body(e_ref, w1_ref, w2_ref, o0_ref, o1_ref):
    e0 = jnp.dot(e_ref[...], w1_ref[...], preferred_element_type=jnp.float32)
    o0_ref[...] = e0
    o1_ref[...] = jnp.dot(e0, w2_ref[...], preferred_element_type=jnp.float32)

  return pl.pallas_call(
      body,
      grid=(E // BE,),
      in_specs=[
          pl.BlockSpec((BE, DE), lambda i: (i, 0)),
          pl.BlockSpec((DE, P), lambda i: (0, 0)),
          pl.BlockSpec((P, OUT), lambda i: (0, 0)),
      ],
      out_specs=[
          pl.BlockSpec((BE, P), lambda i: (i, 0)),
          pl.BlockSpec((BE, OUT), lambda i: (i, 0)),
      ],
      out_shape=[
          jax.ShapeDtypeStruct((E, P), jnp.float32),
          jax.ShapeDtypeStruct((E, OUT), jnp.float32),
      ],
  )(edge_emb, W_edge_prep, W_e_self_0)


def _sc_edge(F, E_self, src_idx, dst_idx):
  E, OUT = E_self.shape
  EC = 128
  n_chunks = E // EC
  iters = (n_chunks + NW - 1) // NW
  nj = OUT // LANES

  @functools.partial(
      pl.kernel,
      out_type=jax.ShapeDtypeStruct((E, OUT), jnp.float32),
      mesh=_sc_mesh(),
      compiler_params=_SC_PARAMS,
      scratch_types=[
          pltpu.VMEM((EC,), jnp.int32),
          pltpu.VMEM((EC,), jnp.int32),
          pltpu.VMEM((EC, OUT), jnp.float32),
          pltpu.VMEM((EC, OUT), jnp.float32),
          pltpu.VMEM((EC, OUT), jnp.float32),
          pltpu.VMEM((EC, OUT), jnp.float32),
          pltpu.SemaphoreType.DMA,
      ],
  )
  def k(f_hbm, es_hbm, src_hbm, dst_hbm, out_hbm,
        idx_s, idx_d, buf_s, buf_d, buf_e, buf_o, sem):
    wid = lax.axis_index("s") * NC + lax.axis_index("c")

    def body(i, carry):
      c = i * NW + wid

      @pl.when(c < n_chunks)
      def _():
        base = c * EC
        pltpu.sync_copy(src_hbm.at[pl.ds(base, EC)], idx_s)
        pltpu.sync_copy(dst_hbm.at[pl.ds(base, EC)], idx_d)
        cp_s = pltpu.async_copy(f_hbm.at[idx_s], buf_s, sem)
        cp_d = pltpu.async_copy(f_hbm.at[idx_d], buf_d, sem)
        cp_e = pltpu.async_copy(es_hbm.at[pl.ds(base, EC)], buf_e, sem)
        cp_s.wait()
        cp_d.wait()
        cp_e.wait()

        def inner(e, icarry):
          for j in range(nj):
            sl = pl.ds(j * LANES, LANES)
            v = buf_e[e, sl] + buf_s[e, sl] + buf_d[e, sl]
            buf_o[e, sl] = jnp.maximum(v, 0.0)
          return icarry

        lax.fori_loop(0, EC, inner, 0)
        pltpu.sync_copy(buf_o, out_hbm.at[pl.ds(base, EC)])

      return carry

    lax.fori_loop(0, iters, body, 0)

  return k(F, E_self, src_idx, dst_idx)


def _sc_nodesum(table, n2e_flat, N, S):
  OUT = table.shape[1]
  GC = 128 // S
  NG = 2
  NODES_PER_CHUNK = GC * NG
  n_chunks = N // NODES_PER_CHUNK
  iters = (n_chunks + NW - 1) // NW
  nj = OUT // LANES

  @functools.partial(
      pl.kernel,
      out_type=jax.ShapeDtypeStruct((N, OUT), jnp.float32),
      mesh=_sc_mesh(),
      compiler_params=_SC_PARAMS,
      scratch_types=[
          pltpu.VMEM((128,), jnp.int32),
          pltpu.VMEM((128,), jnp.int32),
          pltpu.VMEM((NODES_PER_CHUNK * S, OUT), jnp.float32),
          pltpu.VMEM((NODES_PER_CHUNK, OUT), jnp.float32),
          pltpu.SemaphoreType.DMA,
      ],
  )
  def k(tab_hbm, idx_hbm, out_hbm, idx_a, idx_b, buf, buf_o, sem):
    wid = lax.axis_index("s") * NC + lax.axis_index("c")

    def body(i, carry):
      c = i * NW + wid

      @pl.when(c < n_chunks)
      def _():
        ibase = c * (NODES_PER_CHUNK * S)
        pltpu.sync_copy(idx_hbm.at[pl.ds(ibase, 128)], idx_a)
        pltpu.sync_copy(idx_hbm.at[pl.ds(ibase + 128, 128)], idx_b)
        cp_a = pltpu.async_copy(tab_hbm.at[idx_a], buf.at[pl.ds(0, 128)], sem)
        cp_b = pltpu.async_copy(tab_hbm.at[idx_b], buf.at[pl.ds(128, 128)],
                                sem)
        cp_a.wait()
        cp_b.wait()

        def node_body(kk, icarry):
          row = kk * S
          for j in range(nj):
            sl = pl.ds(j * LANES, LANES)
            acc = buf[row, sl]
            for s in range(1, S):
              acc = acc + buf[row + s, sl]
            buf_o[kk, sl] = acc
          return icarry

        lax.fori_loop(0, NODES_PER_CHUNK, node_body, 0)
        pltpu.sync_copy(buf_o, out_hbm.at[pl.ds(c * NODES_PER_CHUNK,
                                                NODES_PER_CHUNK)])

      return carry

    lax.fori_loop(0, iters, body, 0)

  return k(table, n2e_flat)


def _tc_final(id_emb, sum0, sum1, Wns0, Wnn0, Wns1, Wnn1, S):
  N, P = id_emb.shape
  OUT = Wns0.shape[1]
  BN = 2000
  inv_s = 1.0 / S

  def body(id_ref, s0_ref, s1_ref, a0_ref, b0_ref, a1_ref, b1_ref, o_ref):
    m0 = s0_ref[...] * inv_s
    f1 = jnp.maximum(
        jnp.dot(id_ref[...], a0_ref[...], preferred_element_type=jnp.float32)
        + jnp.dot(m0, b0_ref[...], preferred_element_type=jnp.float32), 0.0)
    m1 = s1_ref[...] * inv_s
    f2 = jnp.maximum(
        jnp.dot(f1, a1_ref[...], preferred_element_type=jnp.float32)
        + jnp.dot(m1, b1_ref[...], preferred_element_type=jnp.float32), 0.0)
    o_ref[...] = jnp.concatenate([f1, f2], axis=-1)[None]

  return pl.pallas_call(
      body,
      grid=(N // BN,),
      in_specs=[
          pl.BlockSpec((BN, P), lambda i: (i, 0)),
          pl.BlockSpec((BN, OUT), lambda i: (i, 0)),
          pl.BlockSpec((BN, OUT), lambda i: (i, 0)),
          pl.BlockSpec((P, OUT), lambda i: (0, 0)),
          pl.BlockSpec((OUT, OUT), lambda i: (0, 0)),
          pl.BlockSpec((OUT, OUT), lambda i: (0, 0)),
          pl.BlockSpec((OUT, OUT), lambda i: (0, 0)),
      ],
      out_specs=pl.BlockSpec((1, BN, 2 * OUT), lambda i: (0, i, 0)),
      out_shape=jax.ShapeDtypeStruct((1, N, 2 * OUT), jnp.float32),
  )(id_emb, sum0, sum1, Wns0, Wnn0, Wns1, Wnn1)


def kernel(feats, node2edge_idx, edge_emb, edge_node_adj, id_emb,
           W_prep, W_edge_prep,
           W_e_self_0, W_e_neigh_0, W_n_self_0, W_n_neigh_0,
           W_e_self_1, W_e_neigh_1, W_n_self_1, W_n_neigh_1):
  N, S = node2edge_idx.shape

  src_idx = edge_node_adj[:, 0]
  dst_idx = edge_node_adj[:, 1]
  n2e_flat = node2edge_idx.reshape(-1)

  F = _tc_node_prep(feats, W_prep, W_e_neigh_0)
  edges0, e_self = _tc_edge_prep(edge_emb, W_edge_prep, W_e_self_0)
  edges1 = _sc_edge(F, e_self, src_idx, dst_idx)
  sum0 = _sc_nodesum(edges0, n2e_flat, N, S)
  sum1 = _sc_nodesum(edges1, n2e_flat, N, S)
  return _tc_final(id_emb, sum0, sum1,
                   W_n_self_0, W_n_neigh_0, W_n_self_1, W_n_neigh_1, S)


# speedup vs baseline: 6.5596x; 1.3606x over previous
import functools

import jax
import jax.numpy as jnp
from jax import lax
from jax.experimental import pallas as pl
from jax.experimental.pallas import tpu as pltpu
from jax.experimental.pallas import tpu_sc as plsc

NC = 2
NS = 16
NW = NC * NS
LANES = 16
IDXC = 128


def _sc_mesh():
  return plsc.VectorSubcoreMesh(
      core_axis_name="c", subcore_axis_name="s", num_cores=NC, num_subcores=NS)


_SC_PARAMS = pltpu.CompilerParams(use_tc_tiling_on_sc=False)


def _tc_node_prep(feats, W_prep, W_e_neigh_0):
  N, DF = feats.shape
  P = W_prep.shape[1]
  OUT = W_e_neigh_0.shape[1]
  BN = 2000

  def body(f_ref, wp_ref, wn_ref, o_ref):
    f0 = jnp.dot(f_ref[...], wp_ref[...], preferred_element_type=jnp.float32)
    o_ref[...] = 0.5 * jnp.dot(f0, wn_ref[...],
                               preferred_element_type=jnp.float32)

  return pl.pallas_call(
      body,
      grid=(N // BN,),
      in_specs=[
          pl.BlockSpec((BN, DF), lambda i: (i, 0)),
          pl.BlockSpec((DF, P), lambda i: (0, 0)),
          pl.BlockSpec((P, OUT), lambda i: (0, 0)),
      ],
      out_specs=pl.BlockSpec((BN, OUT), lambda i: (i, 0)),
      out_shape=jax.ShapeDtypeStruct((N, OUT), jnp.float32),
  )(feats, W_prep, W_e_neigh_0)


def _tc_edge_prep(edge_emb, W_edge_prep, W_e_self_0):
  E, DE = edge_emb.shape
  P = W_edge_prep.shape[1]
  OUT = W_e_self_0.shape[1]
  BE = 2000

  def body(e_ref, w1_ref, w2_ref, o_ref):
    e0 = jnp.dot(e_ref[...], w1_ref[...], preferred_element_type=jnp.float32)
    e1 = jnp.dot(e0, w2_ref[...], preferred_element_type=jnp.float32)
    o_ref[...] = jnp.concatenate([e0, e1], axis=-1)

  return pl.pallas_call(
      body,
      grid=(E // BE,),
      in_specs=[
          pl.BlockSpec((BE, DE), lambda i: (i, 0)),
          pl.BlockSpec((DE, P), lambda i: (0, 0)),
          pl.BlockSpec((P, OUT), lambda i: (0, 0)),
      ],
      out_specs=pl.BlockSpec((BE, P + OUT), lambda i: (i, 0)),
      out_shape=jax.ShapeDtypeStruct((E, P + OUT), jnp.float32),
  )(edge_emb, W_edge_prep, W_e_self_0)


def _sc_edge(F, EP, idx2):
  E2 = EP.shape[1]
  E = EP.shape[0]
  OUT = F.shape[1]
  EC = IDXC
  n_chunks = E // EC
  iters = (n_chunks + NW - 1) // NW
  pair_iters = (iters + 1) // 2
  nj = OUT // LANES

  @functools.partial(
      pl.kernel,
      out_type=jax.ShapeDtypeStruct((E, E2), jnp.float32),
      mesh=_sc_mesh(),
      compiler_params=_SC_PARAMS,
      scratch_types=[
          pltpu.VMEM((2 * EC,), jnp.int32),
          pltpu.VMEM((2 * EC,), jnp.int32),
          pltpu.VMEM((2 * EC, OUT), jnp.float32),
          pltpu.VMEM((2 * EC, OUT), jnp.float32),
          pltpu.VMEM((EC, E2), jnp.float32),
          pltpu.VMEM((EC, E2), jnp.float32),
          pltpu.SemaphoreType.DMA,
          pltpu.SemaphoreType.DMA,
          pltpu.SemaphoreType.DMA,
          pltpu.SemaphoreType.DMA,
      ],
  )
  def k(f_hbm, ep_hbm, idx_hbm, out_hbm,
        idx0, idx1, sd0, sd1, ep0, ep1, si0, si1, so0, so1):
    wid = lax.axis_index("s") * NC + lax.axis_index("c")
    idx_b = (idx0, idx1)
    sd_b = (sd0, sd1)
    ep_b = (ep0, ep1)
    si_b = (si0, si1)
    so_b = (so0, so1)

    def fire(slot, c, first):
      if not first:
        @pl.when(c >= 2 * NW)
        def _():
          pltpu.make_async_copy(
              ep_b[slot], out_hbm.at[pl.ds(0, EC)], so_b[slot]).wait()
      base = c * EC
      pltpu.sync_copy(idx_hbm.at[pl.ds(2 * base, 2 * EC)], idx_b[slot])
      pltpu.async_copy(
          f_hbm.at[idx_b[slot].at[pl.ds(0, EC)]],
          sd_b[slot].at[pl.ds(0, EC)], si_b[slot])
      pltpu.async_copy(
          f_hbm.at[idx_b[slot].at[pl.ds(EC, EC)]],
          sd_b[slot].at[pl.ds(EC, EC)], si_b[slot])
      pltpu.async_copy(ep_hbm.at[pl.ds(base, EC)], ep_b[slot], si_b[slot])

    def wait_in(slot):
      pltpu.make_async_copy(
          f_hbm.at[idx_b[slot].at[pl.ds(0, EC)]],
          sd_b[slot].at[pl.ds(0, EC)], si_b[slot]).wait()
      pltpu.make_async_copy(
          f_hbm.at[idx_b[slot].at[pl.ds(EC, EC)]],
          sd_b[slot].at[pl.ds(EC, EC)], si_b[slot]).wait()
      pltpu.make_async_copy(
          ep_hbm.at[pl.ds(0, EC)], ep_b[slot], si_b[slot]).wait()

    def compute_write(slot, c):
      sd = sd_b[slot]
      ep = ep_b[slot]

      def inner(e, icarry):
        for j in range(nj):
          sl = pl.ds(j * LANES, LANES)
          sle = pl.ds(OUT + j * LANES, LANES)
          v = ep[e, sle] + sd[e, sl] + sd[EC + e, sl]
          ep[e, sle] = jnp.maximum(v, 0.0)
        return icarry

      lax.fori_loop(0, EC, inner, 0)
      pltpu.async_copy(ep, out_hbm.at[pl.ds(c * EC, EC)], so_b[slot])

    fire(0, wid, True)

    def body(t, carry):
      i0 = 2 * t
      c0 = i0 * NW + wid
      c1 = c0 + NW
      c2 = c1 + NW

      @pl.when(c0 < n_chunks)
      def _():
        wait_in(0)

      @pl.when(c1 < n_chunks)
      def _():
        fire(1, c1, False)

      @pl.when(c0 < n_chunks)
      def _():
        compute_write(0, c0)

      @pl.when(c1 < n_chunks)
      def _():
        wait_in(1)

      @pl.when(c2 < n_chunks)
      def _():
        fire(0, c2, False)

      @pl.when(c1 < n_chunks)
      def _():
        compute_write(1, c1)

      return carry

    lax.fori_loop(0, pair_iters, body, 0)

    def drain(slot, c):
      @pl.when(c < n_chunks)
      def _():
        pltpu.make_async_copy(
            ep_b[slot], out_hbm.at[pl.ds(0, EC)], so_b[slot]).wait()

    last0 = (iters - 1) if (iters - 1) % 2 == 0 else (iters - 2)
    last1 = (iters - 1) if (iters - 1) % 2 == 1 else (iters - 2)
    if last0 >= 0:
      drain(0, last0 * NW + wid)
    if last1 >= 0:
      drain(1, last1 * NW + wid)

  return k(F, EP, idx2)


def _sc_nodesum(EB, n2e_flat, N, S):
  W = EB.shape[1]
  E = EB.shape[0]
  GC = 2 * IDXC // S
  n_chunks = N // GC
  iters = (n_chunks + NW - 1) // NW
  pair_iters = (iters + 1) // 2
  nj = W // LANES

  @functools.partial(
      pl.kernel,
      out_type=jax.ShapeDtypeStruct((N, W), jnp.float32),
      mesh=_sc_mesh(),
      compiler_params=_SC_PARAMS,
      scratch_types=[
          pltpu.VMEM((2 * IDXC,), jnp.int32),
          pltpu.VMEM((2 * IDXC,), jnp.int32),
          pltpu.VMEM((2 * IDXC, W), jnp.float32),
          pltpu.VMEM((2 * IDXC, W), jnp.float32),
          pltpu.VMEM((GC, W), jnp.float32),
          pltpu.VMEM((GC, W), jnp.float32),
          pltpu.SemaphoreType.DMA,
          pltpu.SemaphoreType.DMA,
          pltpu.SemaphoreType.DMA,
          pltpu.SemaphoreType.DMA,
      ],
  )
  def k(eb_hbm, idx_hbm, out_hbm,
        idx0, idx1, g0, g1, o0, o1, si0, si1, so0, so1):
    wid = lax.axis_index("s") * NC + lax.axis_index("c")
    idx_b = (idx0, idx1)
    g_b = (g0, g1)
    o_b = (o0, o1)
    si_b = (si0, si1)
    so_b = (so0, so1)

    def fire(slot, c):
      base = c * (GC * S)
      pltpu.sync_copy(idx_hbm.at[pl.ds(base, 2 * IDXC)], idx_b[slot])
      pltpu.async_copy(
          eb_hbm.at[idx_b[slot].at[pl.ds(0, IDXC)]],
          g_b[slot].at[pl.ds(0, IDXC)], si_b[slot])
      pltpu.async_copy(
          eb_hbm.at[idx_b[slot].at[pl.ds(IDXC, IDXC)]],
          g_b[slot].at[pl.ds(IDXC, IDXC)], si_b[slot])

    def wait_in(slot):
      pltpu.make_async_copy(
          eb_hbm.at[idx_b[slot].at[pl.ds(0, IDXC)]],
          g_b[slot].at[pl.ds(0, IDXC)], si_b[slot]).wait()
      pltpu.make_async_copy(
          eb_hbm.at[idx_b[slot].at[pl.ds(IDXC, IDXC)]],
          g_b[slot].at[pl.ds(IDXC, IDXC)], si_b[slot]).wait()

    def compute_write(slot, c, first):
      g = g_b[slot]
      o = o_b[slot]
      if not first:
        @pl.when(c >= 2 * NW)
        def _():
          pltpu.make_async_copy(
              o, out_hbm.at[pl.ds(0, GC)], so_b[slot]).wait()

      def node_body(kk, icarry):
        row = kk * S
        for j in range(nj):
          sl = pl.ds(j * LANES, LANES)
          acc = g[row, sl]
          for s in range(1, S):
            acc = acc + g[row + s, sl]
          o[kk, sl] = acc
        return icarry

      lax.fori_loop(0, GC, node_body, 0)
      pltpu.async_copy(o, out_hbm.at[pl.ds(c * GC, GC)], so_b[slot])

    fire(0, wid)

    def body(t, carry):
      i0 = 2 * t
      c0 = i0 * NW + wid
      c1 = c0 + NW
      c2 = c1 + NW

      @pl.when(c0 < n_chunks)
      def _():
        wait_in(0)

      @pl.when(c1 < n_chunks)
      def _():
        fire(1, c1)

      @pl.when(c0 < n_chunks)
      def _():
        compute_write(0, c0, False)

      @pl.when(c1 < n_chunks)
      def _():
        wait_in(1)

      @pl.when(c2 < n_chunks)
      def _():
        fire(0, c2)

      @pl.when(c1 < n_chunks)
      def _():
        compute_write(1, c1, False)

      return carry

    lax.fori_loop(0, pair_iters, body, 0)

    def drain(slot, c):
      @pl.when(c < n_chunks)
      def _():
        pltpu.make_async_copy(
            o_b[slot], out_hbm.at[pl.ds(0, GC)], so_b[slot]).wait()

    last0 = (iters - 1) if (iters - 1) % 2 == 0 else (iters - 2)
    last1 = (iters - 1) if (iters - 1) % 2 == 1 else (iters - 2)
    if last0 >= 0:
      drain(0, last0 * NW + wid)
    if last1 >= 0:
      drain(1, last1 * NW + wid)

  return k(EB, n2e_flat)


def _tc_final(id_emb, sums, Wns0, Wnn0, Wns1, Wnn1, S):
  N, P = id_emb.shape
  OUT = Wns0.shape[1]
  BN = 2000
  inv_s = 1.0 / S

  def body(id_ref, s_ref, a0_ref, b0_ref, a1_ref, b1_ref, o_ref):
    m0 = s_ref[:, :OUT] * inv_s
    f1 = jnp.maximum(
        jnp.dot(id_ref[...], a0_ref[...], preferred_element_type=jnp.float32)
        + jnp.dot(m0, b0_ref[...], preferred_element_type=jnp.float32), 0.0)
    m1 = s_ref[:, OUT:] * inv_s
    f2 = jnp.maximum(
        jnp.dot(f1, a1_ref[...], preferred_element_type=jnp.float32)
        + jnp.dot(m1, b1_ref[...], preferred_element_type=jnp.float32), 0.0)
    o_ref[...] = jnp.concatenate([f1, f2], axis=-1)[None]

  return pl.pallas_call(
      body,
      grid=(N // BN,),
      in_specs=[
          pl.BlockSpec((BN, P), lambda i: (i, 0)),
          pl.BlockSpec((BN, 2 * OUT), lambda i: (i, 0)),
          pl.BlockSpec((P, OUT), lambda i: (0, 0)),
          pl.BlockSpec((OUT, OUT), lambda i: (0, 0)),
          pl.BlockSpec((OUT, OUT), lambda i: (0, 0)),
          pl.BlockSpec((OUT, OUT), lambda i: (0, 0)),
      ],
      out_specs=pl.BlockSpec((1, BN, 2 * OUT), lambda i: (0, i, 0)),
      out_shape=jax.ShapeDtypeStruct((1, N, 2 * OUT), jnp.float32),
  )(id_emb, sums, Wns0, Wnn0, Wns1, Wnn1)


def kernel(feats, node2edge_idx, edge_emb, edge_node_adj, id_emb,
           W_prep, W_edge_prep,
           W_e_self_0, W_e_neigh_0, W_n_self_0, W_n_neigh_0,
           W_e_self_1, W_e_neigh_1, W_n_self_1, W_n_neigh_1):
  N, S = node2edge_idx.shape
  E = edge_emb.shape[0]

  src = edge_node_adj[:, 0].reshape(-1, IDXC)
  dst = edge_node_adj[:, 1].reshape(-1, IDXC)
  idx2 = jnp.concatenate([src, dst], axis=1).reshape(-1)
  n2e_flat = node2edge_idx.reshape(-1)

  F = _tc_node_prep(feats, W_prep, W_e_neigh_0)
  EP = _tc_edge_prep(edge_emb, W_edge_prep, W_e_self_0)
  EB = _sc_edge(F, EP, idx2)
  sums = _sc_nodesum(EB, n2e_flat, N, S)
  return _tc_final(id_emb, sums,
                   W_n_self_0, W_n_neigh_0, W_n_self_1, W_n_neigh_1, S)


# speedup vs baseline: 7.0326x; 1.0721x over previous
import functools

import jax
import jax.numpy as jnp
from jax import lax
from jax.experimental import pallas as pl
from jax.experimental.pallas import tpu as pltpu
from jax.experimental.pallas import tpu_sc as plsc

NC = 2
NS = 16
NW = NC * NS
LANES = 16
IDXC = 128


def _sc_mesh():
  return plsc.VectorSubcoreMesh(
      core_axis_name="c", subcore_axis_name="s", num_cores=NC, num_subcores=NS)


_SC_PARAMS = pltpu.CompilerParams(use_tc_tiling_on_sc=False)


def _tc_node_prep(feats, W_prep, W_e_neigh_0):
  N, DF = feats.shape
  P = W_prep.shape[1]
  OUT = W_e_neigh_0.shape[1]
  BN = 2000

  def body(f_ref, wp_ref, wn_ref, o_ref):
    f0 = jnp.dot(f_ref[...], wp_ref[...], preferred_element_type=jnp.float32)
    o_ref[...] = 0.5 * jnp.dot(f0, wn_ref[...],
                               preferred_element_type=jnp.float32)

  return pl.pallas_call(
      body,
      grid=(N // BN,),
      in_specs=[
          pl.BlockSpec((BN, DF), lambda i: (i, 0)),
          pl.BlockSpec((DF, P), lambda i: (0, 0)),
          pl.BlockSpec((P, OUT), lambda i: (0, 0)),
      ],
      out_specs=pl.BlockSpec((BN, OUT), lambda i: (i, 0)),
      out_shape=jax.ShapeDtypeStruct((N, OUT), jnp.float32),
  )(feats, W_prep, W_e_neigh_0)


def _tc_edge_prep(edge_emb, W_edge_prep, W_e_self_0):
  E, DE = edge_emb.shape
  P = W_edge_prep.shape[1]
  OUT = W_e_self_0.shape[1]
  BE = 2000

  def body(e_ref, w1_ref, w2_ref, o_ref):
    e0 = jnp.dot(e_ref[...], w1_ref[...], preferred_element_type=jnp.float32)
    e1 = jnp.dot(e0, w2_ref[...], preferred_element_type=jnp.float32)
    o_ref[...] = jnp.concatenate([e0, e1], axis=-1)

  return pl.pallas_call(
      body,
      grid=(E // BE,),
      in_specs=[
          pl.BlockSpec((BE, DE), lambda i: (i, 0)),
          pl.BlockSpec((DE, P), lambda i: (0, 0)),
          pl.BlockSpec((P, OUT), lambda i: (0, 0)),
      ],
      out_specs=pl.BlockSpec((BE, P + OUT), lambda i: (i, 0)),
      out_shape=jax.ShapeDtypeStruct((E, P + OUT), jnp.float32),
  )(edge_emb, W_edge_prep, W_e_self_0)


def _sc_edge(F, EP, idx2):
  E2 = EP.shape[1]
  E = EP.shape[0]
  OUT = F.shape[1]
  EC = IDXC
  n_chunks = E // EC
  ipw = (n_chunks + NW - 1) // NW
  titers = (ipw + 2) // 3
  nj = OUT // LANES

  @functools.partial(
      pl.kernel,
      out_type=jax.ShapeDtypeStruct((E, E2), jnp.float32),
      mesh=_sc_mesh(),
      compiler_params=_SC_PARAMS,
      scratch_types=[
          pltpu.VMEM((ipw * 2 * EC,), jnp.int32),
          pltpu.VMEM((2 * EC, OUT), jnp.float32),
          pltpu.VMEM((2 * EC, OUT), jnp.float32),
          pltpu.VMEM((2 * EC, OUT), jnp.float32),
          pltpu.VMEM((EC, E2), jnp.float32),
          pltpu.VMEM((EC, E2), jnp.float32),
          pltpu.VMEM((EC, E2), jnp.float32),
          pltpu.SemaphoreType.DMA,
          pltpu.SemaphoreType.DMA,
          pltpu.SemaphoreType.DMA,
          pltpu.SemaphoreType.DMA,
          pltpu.SemaphoreType.DMA,
          pltpu.SemaphoreType.DMA,
      ],
  )
  def k(f_hbm, ep_hbm, idx_hbm, out_hbm,
        idx_all, sd0, sd1, sd2, ep0, ep1, ep2,
        si0, si1, si2, so0, so1, so2):
    wid = lax.axis_index("s") * NC + lax.axis_index("c")
    cbase = wid * ipw
    sd_b = (sd0, sd1, sd2)
    ep_b = (ep0, ep1, ep2)
    si_b = (si0, si1, si2)
    so_b = (so0, so1, so2)

    pltpu.sync_copy(idx_hbm.at[pl.ds(cbase * 2 * EC, ipw * 2 * EC)], idx_all)

    def cond(i):
      return (i < ipw) & (cbase + i < n_chunks)

    def fire(slot, i, first):
      @pl.when(cond(i))
      def _():
        if not first:
          @pl.when(i >= 3)
          def _():
            pltpu.make_async_copy(
                ep_b[slot], out_hbm.at[pl.ds(0, EC)], so_b[slot]).wait()
        off = i * 2 * EC
        pltpu.async_copy(
            f_hbm.at[idx_all.at[pl.ds(off, EC)]],
            sd_b[slot].at[pl.ds(0, EC)], si_b[slot])
        pltpu.async_copy(
            f_hbm.at[idx_all.at[pl.ds(off + EC, EC)]],
            sd_b[slot].at[pl.ds(EC, EC)], si_b[slot])
        pltpu.async_copy(
            ep_hbm.at[pl.ds((cbase + i) * EC, EC)], ep_b[slot], si_b[slot])

    def process(slot, i):
      @pl.when(cond(i))
      def _():
        pltpu.make_async_copy(
            f_hbm.at[idx_all.at[pl.ds(0, EC)]],
            sd_b[slot].at[pl.ds(0, EC)], si_b[slot]).wait()
        pltpu.make_async_copy(
            f_hbm.at[idx_all.at[pl.ds(0, EC)]],
            sd_b[slot].at[pl.ds(EC, EC)], si_b[slot]).wait()
        pltpu.make_async_copy(
            ep_hbm.at[pl.ds(0, EC)], ep_b[slot], si_b[slot]).wait()
        sd = sd_b[slot]
        ep = ep_b[slot]

        def inner(e, icarry):
          for j in range(nj):
            sl = pl.ds(j * LANES, LANES)
            sle = pl.ds(OUT + j * LANES, LANES)
            v = ep[e, sle] + sd[e, sl] + sd[EC + e, sl]
            ep[e, sle] = jnp.maximum(v, 0.0)
          return icarry

        lax.fori_loop(0, EC, inner, 0, unroll=2)
        pltpu.async_copy(
            ep, out_hbm.at[pl.ds((cbase + i) * EC, EC)], so_b[slot])

    fire(0, 0, True)
    fire(1, 1, True)

    def body(t, carry):
      i0 = 3 * t
      fire(2, i0 + 2, False)
      process(0, i0)
      fire(0, i0 + 3, False)
      process(1, i0 + 1)
      fire(1, i0 + 4, False)
      process(2, i0 + 2)
      return carry

    lax.fori_loop(0, titers, body, 0)

    for slot in range(3):
      @pl.when(cond(slot))
      def _():
        pltpu.make_async_copy(
            ep_b[slot], out_hbm.at[pl.ds(0, EC)], so_b[slot]).wait()

  return k(F, EP, idx2)


def _sc_nodesum(EB, n2e_flat, N, S):
  W = EB.shape[1]
  E = EB.shape[0]
  GC = 2 * IDXC // S
  n_chunks = N // GC
  ipw = (n_chunks + NW - 1) // NW
  titers = (ipw + 2) // 3
  nj = W // LANES

  @functools.partial(
      pl.kernel,
      out_type=jax.ShapeDtypeStruct((N, W), jnp.float32),
      mesh=_sc_mesh(),
      compiler_params=_SC_PARAMS,
      scratch_types=[
          pltpu.VMEM((ipw * 2 * IDXC,), jnp.int32),
          pltpu.VMEM((2 * IDXC, W), jnp.float32),
          pltpu.VMEM((2 * IDXC, W), jnp.float32),
          pltpu.VMEM((2 * IDXC, W), jnp.float32),
          pltpu.VMEM((GC, W), jnp.float32),
          pltpu.VMEM((GC, W), jnp.float32),
          pltpu.VMEM((GC, W), jnp.float32),
          pltpu.SemaphoreType.DMA,
          pltpu.SemaphoreType.DMA,
          pltpu.SemaphoreType.DMA,
          pltpu.SemaphoreType.DMA,
          pltpu.SemaphoreType.DMA,
          pltpu.SemaphoreType.DMA,
      ],
  )
  def k(eb_hbm, idx_hbm, out_hbm,
        idx_all, g0, g1, g2, o0, o1, o2, si0, si1, si2, so0, so1, so2):
    wid = lax.axis_index("s") * NC + lax.axis_index("c")
    cbase = wid * ipw
    g_b = (g0, g1, g2)
    o_b = (o0, o1, o2)
    si_b = (si0, si1, si2)
    so_b = (so0, so1, so2)

    pltpu.sync_copy(
        idx_hbm.at[pl.ds(cbase * 2 * IDXC, ipw * 2 * IDXC)], idx_all)

    def cond(i):
      return (i < ipw) & (cbase + i < n_chunks)

    def fire(slot, i):
      @pl.when(cond(i))
      def _():
        off = i * 2 * IDXC
        pltpu.async_copy(
            eb_hbm.at[idx_all.at[pl.ds(off, IDXC)]],
            g_b[slot].at[pl.ds(0, IDXC)], si_b[slot])
        pltpu.async_copy(
            eb_hbm.at[idx_all.at[pl.ds(off + IDXC, IDXC)]],
            g_b[slot].at[pl.ds(IDXC, IDXC)], si_b[slot])

    def process(slot, i, first):
      @pl.when(cond(i))
      def _():
        pltpu.make_async_copy(
            eb_hbm.at[idx_all.at[pl.ds(0, IDXC)]],
            g_b[slot].at[pl.ds(0, IDXC)], si_b[slot]).wait()
        pltpu.make_async_copy(
            eb_hbm.at[idx_all.at[pl.ds(0, IDXC)]],
            g_b[slot].at[pl.ds(IDXC, IDXC)], si_b[slot]).wait()
        if not first:
          @pl.when(i >= 3)
          def _():
            pltpu.make_async_copy(
                o_b[slot], out_hbm.at[pl.ds(0, GC)], so_b[slot]).wait()
        g = g_b[slot]
        o = o_b[slot]

        def node_body(kk, icarry):
          row = kk * S
          for j in range(nj):
            sl = pl.ds(j * LANES, LANES)
            acc = g[row, sl]
            for s in range(1, S):
              acc = acc + g[row + s, sl]
            o[kk, sl] = acc
          return icarry

        lax.fori_loop(0, GC, node_body, 0)
        pltpu.async_copy(
            o, out_hbm.at[pl.ds((cbase + i) * GC, GC)], so_b[slot])

    fire(0, 0)
    fire(1, 1)

    def body(t, carry):
      i0 = 3 * t
      fire(2, i0 + 2)
      process(0, i0, False)
      fire(0, i0 + 3)
      process(1, i0 + 1, False)
      fire(1, i0 + 4)
      process(2, i0 + 2, False)
      return carry

    lax.fori_loop(0, titers, body, 0)

    for slot in range(3):
      @pl.when(cond(slot))
      def _():
        pltpu.make_async_copy(
            o_b[slot], out_hbm.at[pl.ds(0, GC)], so_b[slot]).wait()

  return k(EB, n2e_flat)


def _tc_final(id_emb, sums, Wns0, Wnn0, Wns1, Wnn1, S):
  N, P = id_emb.shape
  OUT = Wns0.shape[1]
  BN = 2000
  inv_s = 1.0 / S

  def body(id_ref, s_ref, a0_ref, b0_ref, a1_ref, b1_ref, o_ref):
    m0 = s_ref[:, :OUT] * inv_s
    f1 = jnp.maximum(
        jnp.dot(id_ref[...], a0_ref[...], preferred_element_type=jnp.float32)
        + jnp.dot(m0, b0_ref[...], preferred_element_type=jnp.float32), 0.0)
    m1 = s_ref[:, OUT:] * inv_s
    f2 = jnp.maximum(
        jnp.dot(f1, a1_ref[...], preferred_element_type=jnp.float32)
        + jnp.dot(m1, b1_ref[...], preferred_element_type=jnp.float32), 0.0)
    o_ref[...] = jnp.concatenate([f1, f2], axis=-1)[None]

  return pl.pallas_call(
      body,
      grid=(N // BN,),
      in_specs=[
          pl.BlockSpec((BN, P), lambda i: (i, 0)),
          pl.BlockSpec((BN, 2 * OUT), lambda i: (i, 0)),
          pl.BlockSpec((P, OUT), lambda i: (0, 0)),
          pl.BlockSpec((OUT, OUT), lambda i: (0, 0)),
          pl.BlockSpec((OUT, OUT), lambda i: (0, 0)),
          pl.BlockSpec((OUT, OUT), lambda i: (0, 0)),
      ],
      out_specs=pl.BlockSpec((1, BN, 2 * OUT), lambda i: (0, i, 0)),
      out_shape=jax.ShapeDtypeStruct((1, N, 2 * OUT), jnp.float32),
  )(id_emb, sums, Wns0, Wnn0, Wns1, Wnn1)


def kernel(feats, node2edge_idx, edge_emb, edge_node_adj, id_emb,
           W_prep, W_edge_prep,
           W_e_self_0, W_e_neigh_0, W_n_self_0, W_n_neigh_0,
           W_e_self_1, W_e_neigh_1, W_n_self_1, W_n_neigh_1):
  N, S = node2edge_idx.shape
  E = edge_emb.shape[0]

  src = edge_node_adj[:, 0].reshape(-1, IDXC)
  dst = edge_node_adj[:, 1].reshape(-1, IDXC)
  idx2 = jnp.concatenate([src, dst], axis=1).reshape(-1)
  ipw_e = (E // IDXC + NW - 1) // NW
  idx2 = jnp.pad(idx2, (0, NW * ipw_e * 2 * IDXC - idx2.shape[0]))
  n2e_flat = node2edge_idx.reshape(-1)
  ipw_n = (N * S // (2 * IDXC) + NW - 1) // NW
  n2e_flat = jnp.pad(n2e_flat, (0, NW * ipw_n * 2 * IDXC - n2e_flat.shape[0]))

  F = _tc_node_prep(feats, W_prep, W_e_neigh_0)
  EP = _tc_edge_prep(edge_emb, W_edge_prep, W_e_self_0)
  EB = _sc_edge(F, EP, idx2)
  sums = _sc_nodesum(EB, n2e_flat, N, S)
  return _tc_final(id_emb, sums,
                   W_n_self_0, W_n_neigh_0, W_n_self_1, W_n_neigh_1, S)


# speedup vs baseline: 7.8699x; 1.1190x over previous
import functools

import jax
import jax.numpy as jnp
from jax import lax
from jax.experimental import pallas as pl
from jax.experimental.pallas import tpu as pltpu
from jax.experimental.pallas import tpu_sc as plsc

NC = 2
NS = 16
NW = NC * NS
LANES = 16
IDXC = 128


def _sc_mesh():
  return plsc.VectorSubcoreMesh(
      core_axis_name="c", subcore_axis_name="s", num_cores=NC, num_subcores=NS)


_SC_PARAMS = pltpu.CompilerParams(use_tc_tiling_on_sc=False)


def _tc_node_prep(feats, W_prep, W_e_neigh_0):
  N, DF = feats.shape
  P = W_prep.shape[1]
  OUT = W_e_neigh_0.shape[1]
  BN = 2000

  def body(f_ref, wp_ref, wn_ref, o_ref):
    f0 = jnp.dot(f_ref[...], wp_ref[...], preferred_element_type=jnp.float32)
    o_ref[...] = 0.5 * jnp.dot(f0, wn_ref[...],
                               preferred_element_type=jnp.float32)

  return pl.pallas_call(
      body,
      grid=(N // BN,),
      in_specs=[
          pl.BlockSpec((BN, DF), lambda i: (i, 0)),
          pl.BlockSpec((DF, P), lambda i: (0, 0)),
          pl.BlockSpec((P, OUT), lambda i: (0, 0)),
      ],
      out_specs=pl.BlockSpec((BN, OUT), lambda i: (i, 0)),
      out_shape=jax.ShapeDtypeStruct((N, OUT), jnp.float32),
  )(feats, W_prep, W_e_neigh_0)


def _tc_edge_prep(edge_emb, W_edge_prep, W_e_self_0):
  E, DE = edge_emb.shape
  P = W_edge_prep.shape[1]
  OUT = W_e_self_0.shape[1]
  BE = 8000

  def body(e_ref, w1_ref, w2_ref, o_ref):
    e0 = jnp.dot(e_ref[...], w1_ref[...], preferred_element_type=jnp.float32)
    e1 = jnp.dot(e0, w2_ref[...], preferred_element_type=jnp.float32)
    o_ref[...] = jnp.concatenate([e0, e1], axis=-1)

  return pl.pallas_call(
      body,
      grid=(E // BE,),
      in_specs=[
          pl.BlockSpec((BE, DE), lambda i: (i, 0)),
          pl.BlockSpec((DE, P), lambda i: (0, 0)),
          pl.BlockSpec((P, OUT), lambda i: (0, 0)),
      ],
      out_specs=pl.BlockSpec((BE, P + OUT), lambda i: (i, 0)),
      out_shape=jax.ShapeDtypeStruct((E, P + OUT), jnp.float32),
  )(edge_emb, W_edge_prep, W_e_self_0)


def _sc_edge(F, EP, idx2):
  E2 = EP.shape[1]
  E = EP.shape[0]
  OUT = F.shape[1]
  EC = IDXC
  n_chunks = E // EC
  ipw = (n_chunks + NW - 1) // NW
  titers = (ipw + 2) // 3
  nj = OUT // LANES

  @functools.partial(
      pl.kernel,
      out_type=jax.ShapeDtypeStruct((E, E2), jnp.float32),
      mesh=_sc_mesh(),
      compiler_params=_SC_PARAMS,
      scratch_types=[
          pltpu.VMEM((ipw * 2 * EC,), jnp.int32),
          pltpu.VMEM((2 * EC, OUT), jnp.float32),
          pltpu.VMEM((2 * EC, OUT), jnp.float32),
          pltpu.VMEM((2 * EC, OUT), jnp.float32),
          pltpu.VMEM((EC, E2), jnp.float32),
          pltpu.VMEM((EC, E2), jnp.float32),
          pltpu.VMEM((EC, E2), jnp.float32),
          pltpu.SemaphoreType.DMA,
          pltpu.SemaphoreType.DMA,
          pltpu.SemaphoreType.DMA,
          pltpu.SemaphoreType.DMA,
          pltpu.SemaphoreType.DMA,
          pltpu.SemaphoreType.DMA,
      ],
  )
  def k(f_hbm, ep_hbm, idx_hbm, out_hbm,
        idx_all, sd0, sd1, sd2, ep0, ep1, ep2,
        si0, si1, si2, so0, so1, so2):
    wid = lax.axis_index("s") * NC + lax.axis_index("c")
    cbase = wid * ipw
    sd_b = (sd0, sd1, sd2)
    ep_b = (ep0, ep1, ep2)
    si_b = (si0, si1, si2)
    so_b = (so0, so1, so2)

    pltpu.sync_copy(idx_hbm.at[pl.ds(cbase * 2 * EC, ipw * 2 * EC)], idx_all)

    def cond(i):
      return (i < ipw) & (cbase + i < n_chunks)

    def fire(slot, i, first):
      @pl.when(cond(i))
      def _():
        if not first:
          @pl.when(i >= 3)
          def _():
            pltpu.make_async_copy(
                ep_b[slot], out_hbm.at[pl.ds(0, EC)], so_b[slot]).wait()
        off = i * 2 * EC
        pltpu.async_copy(
            f_hbm.at[idx_all.at[pl.ds(off, EC)]],
            sd_b[slot].at[pl.ds(0, EC)], si_b[slot])
        pltpu.async_copy(
            f_hbm.at[idx_all.at[pl.ds(off + EC, EC)]],
            sd_b[slot].at[pl.ds(EC, EC)], si_b[slot])
        pltpu.async_copy(
            ep_hbm.at[pl.ds((cbase + i) * EC, EC)], ep_b[slot], si_b[slot])

    def process(slot, i):
      @pl.when(cond(i))
      def _():
        pltpu.make_async_copy(
            f_hbm.at[idx_all.at[pl.ds(0, EC)]],
            sd_b[slot].at[pl.ds(0, EC)], si_b[slot]).wait()
        pltpu.make_async_copy(
            f_hbm.at[idx_all.at[pl.ds(0, EC)]],
            sd_b[slot].at[pl.ds(EC, EC)], si_b[slot]).wait()
        pltpu.make_async_copy(
            ep_hbm.at[pl.ds(0, EC)], ep_b[slot], si_b[slot]).wait()
        sd = sd_b[slot]
        ep = ep_b[slot]

        def inner(e, icarry):
          for j in range(nj):
            sl = pl.ds(j * LANES, LANES)
            sle = pl.ds(OUT + j * LANES, LANES)
            v = ep[e, sle] + sd[e, sl] + sd[EC + e, sl]
            ep[e, sle] = jnp.maximum(v, 0.0)
          return icarry

        lax.fori_loop(0, EC, inner, 0, unroll=2)
        pltpu.async_copy(
            ep, out_hbm.at[pl.ds((cbase + i) * EC, EC)], so_b[slot])

    fire(0, 0, True)
    fire(1, 1, True)

    def body(t, carry):
      i0 = 3 * t
      fire(2, i0 + 2, False)
      process(0, i0)
      fire(0, i0 + 3, False)
      process(1, i0 + 1)
      fire(1, i0 + 4, False)
      process(2, i0 + 2)
      return carry

    lax.fori_loop(0, titers, body, 0)

    for slot in range(3):
      @pl.when(cond(slot))
      def _():
        pltpu.make_async_copy(
            ep_b[slot], out_hbm.at[pl.ds(0, EC)], so_b[slot]).wait()

  return k(F, EP, idx2)


def _sc_nodesum(EB, n2e_flat, N, S):
  W = EB.shape[1]
  E = EB.shape[0]
  GC = 2 * IDXC // S
  n_chunks = N // GC
  ipw = (n_chunks + NW - 1) // NW
  titers = (ipw + 2) // 3
  nj = W // LANES

  @functools.partial(
      pl.kernel,
      out_type=jax.ShapeDtypeStruct((N, W), jnp.float32),
      mesh=_sc_mesh(),
      compiler_params=_SC_PARAMS,
      scratch_types=[
          pltpu.VMEM((ipw * 2 * IDXC,), jnp.int32),
          pltpu.VMEM((2 * IDXC, W), jnp.float32),
          pltpu.VMEM((2 * IDXC, W), jnp.float32),
          pltpu.VMEM((2 * IDXC, W), jnp.float32),
          pltpu.VMEM((GC, W), jnp.float32),
          pltpu.VMEM((GC, W), jnp.float32),
          pltpu.VMEM((GC, W), jnp.float32),
          pltpu.SemaphoreType.DMA,
          pltpu.SemaphoreType.DMA,
          pltpu.SemaphoreType.DMA,
          pltpu.SemaphoreType.DMA,
          pltpu.SemaphoreType.DMA,
          pltpu.SemaphoreType.DMA,
      ],
  )
  def k(eb_hbm, idx_hbm, out_hbm,
        idx_all, g0, g1, g2, o0, o1, o2, si0, si1, si2, so0, so1, so2):
    wid = lax.axis_index("s") * NC + lax.axis_index("c")
    cbase = wid * ipw
    g_b = (g0, g1, g2)
    o_b = (o0, o1, o2)
    si_b = (si0, si1, si2)
    so_b = (so0, so1, so2)

    pltpu.sync_copy(
        idx_hbm.at[pl.ds(cbase * 2 * IDXC, ipw * 2 * IDXC)], idx_all)

    def cond(i):
      return (i < ipw) & (cbase + i < n_chunks)

    def fire(slot, i):
      @pl.when(cond(i))
      def _():
        off = i * 2 * IDXC
        pltpu.async_copy(
            eb_hbm.at[idx_all.at[pl.ds(off, IDXC)]],
            g_b[slot].at[pl.ds(0, IDXC)], si_b[slot])
        pltpu.async_copy(
            eb_hbm.at[idx_all.at[pl.ds(off + IDXC, IDXC)]],
            g_b[slot].at[pl.ds(IDXC, IDXC)], si_b[slot])

    def process(slot, i, first):
      @pl.when(cond(i))
      def _():
        pltpu.make_async_copy(
            eb_hbm.at[idx_all.at[pl.ds(0, IDXC)]],
            g_b[slot].at[pl.ds(0, IDXC)], si_b[slot]).wait()
        pltpu.make_async_copy(
            eb_hbm.at[idx_all.at[pl.ds(0, IDXC)]],
            g_b[slot].at[pl.ds(IDXC, IDXC)], si_b[slot]).wait()
        if not first:
          @pl.when(i >= 3)
          def _():
            pltpu.make_async_copy(
                o_b[slot], out_hbm.at[pl.ds(0, GC)], so_b[slot]).wait()
        g = g_b[slot]
        o = o_b[slot]

        def node_body(kk, icarry):
          row = kk * S
          for j in range(nj):
            sl = pl.ds(j * LANES, LANES)
            acc = g[row, sl]
            for s in range(1, S):
              acc = acc + g[row + s, sl]
            o[kk, sl] = acc
          return icarry

        lax.fori_loop(0, GC, node_body, 0)
        pltpu.async_copy(
            o, out_hbm.at[pl.ds((cbase + i) * GC, GC)], so_b[slot])

    fire(0, 0)
    fire(1, 1)

    def body(t, carry):
      i0 = 3 * t
      fire(2, i0 + 2)
      process(0, i0, False)
      fire(0, i0 + 3)
      process(1, i0 + 1, False)
      fire(1, i0 + 4)
      process(2, i0 + 2, False)
      return carry

    lax.fori_loop(0, titers, body, 0)

    for slot in range(3):
      @pl.when(cond(slot))
      def _():
        pltpu.make_async_copy(
            o_b[slot], out_hbm.at[pl.ds(0, GC)], so_b[slot]).wait()

  return k(EB, n2e_flat)


def _tc_final(id_emb, sums, Wns0, Wnn0, Wns1, Wnn1, S):
  N, P = id_emb.shape
  OUT = Wns0.shape[1]
  BN = 2000
  inv_s = 1.0 / S

  def body(id_ref, s_ref, a0_ref, b0_ref, a1_ref, b1_ref, o_ref):
    m0 = s_ref[:, :OUT] * inv_s
    f1 = jnp.maximum(
        jnp.dot(id_ref[...], a0_ref[...], preferred_element_type=jnp.float32)
        + jnp.dot(m0, b0_ref[...], preferred_element_type=jnp.float32), 0.0)
    m1 = s_ref[:, OUT:] * inv_s
    f2 = jnp.maximum(
        jnp.dot(f1, a1_ref[...], preferred_element_type=jnp.float32)
        + jnp.dot(m1, b1_ref[...], preferred_element_type=jnp.float32), 0.0)
    o_ref[...] = jnp.concatenate([f1, f2], axis=-1)[None]

  return pl.pallas_call(
      body,
      grid=(N // BN,),
      in_specs=[
          pl.BlockSpec((BN, P), lambda i: (i, 0)),
          pl.BlockSpec((BN, 2 * OUT), lambda i: (i, 0)),
          pl.BlockSpec((P, OUT), lambda i: (0, 0)),
          pl.BlockSpec((OUT, OUT), lambda i: (0, 0)),
          pl.BlockSpec((OUT, OUT), lambda i: (0, 0)),
          pl.BlockSpec((OUT, OUT), lambda i: (0, 0)),
      ],
      out_specs=pl.BlockSpec((1, BN, 2 * OUT), lambda i: (0, i, 0)),
      out_shape=jax.ShapeDtypeStruct((1, N, 2 * OUT), jnp.float32),
  )(id_emb, sums, Wns0, Wnn0, Wns1, Wnn1)


def kernel(feats, node2edge_idx, edge_emb, edge_node_adj, id_emb,
           W_prep, W_edge_prep,
           W_e_self_0, W_e_neigh_0, W_n_self_0, W_n_neigh_0,
           W_e_self_1, W_e_neigh_1, W_n_self_1, W_n_neigh_1):
  N, S = node2edge_idx.shape
  E = edge_emb.shape[0]

  src = edge_node_adj[:, 0].reshape(-1, IDXC)
  dst = edge_node_adj[:, 1].reshape(-1, IDXC)
  idx2 = jnp.concatenate([src, dst], axis=1).reshape(-1)
  ipw_e = (E // IDXC + NW - 1) // NW
  idx2 = jnp.pad(idx2, (0, NW * ipw_e * 2 * IDXC - idx2.shape[0]))
  n2e_flat = node2edge_idx.reshape(-1)
  ipw_n = (N * S // (2 * IDXC) + NW - 1) // NW
  n2e_flat = jnp.pad(n2e_flat, (0, NW * ipw_n * 2 * IDXC - n2e_flat.shape[0]))

  F = _tc_node_prep(feats, W_prep, W_e_neigh_0)
  EP = _tc_edge_prep(edge_emb, W_edge_prep, W_e_self_0)
  EB = _sc_edge(F, EP, idx2)
  sums = _sc_nodesum(EB, n2e_flat, N, S)
  return _tc_final(id_emb, sums,
                   W_n_self_0, W_n_neigh_0, W_n_self_1, W_n_neigh_1, S)


# speedup vs baseline: 8.0267x; 1.0199x over previous
import functools

import jax
import jax.numpy as jnp
from jax import lax
from jax.experimental import pallas as pl
from jax.experimental.pallas import tpu as pltpu
from jax.experimental.pallas import tpu_sc as plsc

NC = 2
NS = 16
NW = NC * NS
LANES = 16
IDXC = 128


def _sc_mesh():
  return plsc.VectorSubcoreMesh(
      core_axis_name="c", subcore_axis_name="s", num_cores=NC, num_subcores=NS)


_SC_PARAMS = pltpu.CompilerParams(use_tc_tiling_on_sc=False)


def _tc_node_prep(feats, W_prep, W_e_neigh_0):
  N, DF = feats.shape
  P = W_prep.shape[1]
  OUT = W_e_neigh_0.shape[1]
  BN = 2000

  def body(f_ref, wp_ref, wn_ref, o_ref):
    f0 = jnp.dot(f_ref[...], wp_ref[...], preferred_element_type=jnp.float32)
    o_ref[...] = 0.5 * jnp.dot(f0, wn_ref[...],
                               preferred_element_type=jnp.float32)

  return pl.pallas_call(
      body,
      grid=(N // BN,),
      in_specs=[
          pl.BlockSpec((BN, DF), lambda i: (i, 0)),
          pl.BlockSpec((DF, P), lambda i: (0, 0)),
          pl.BlockSpec((P, OUT), lambda i: (0, 0)),
      ],
      out_specs=pl.BlockSpec((BN, OUT), lambda i: (i, 0)),
      out_shape=jax.ShapeDtypeStruct((N, OUT), jnp.float32),
  )(feats, W_prep, W_e_neigh_0)


def _tc_edge_prep(edge_emb8, W_edge_prep, W_e_self_0):
  E8, DE8 = edge_emb8.shape
  DE = DE8 // 8
  P = W_edge_prep.shape[1]
  OUT = W_e_self_0.shape[1]
  W2 = P + OUT
  BP = 1000

  def body(e_ref, w1_ref, w2_ref, o_ref):
    u = jnp.dot(w1_ref[...], w2_ref[...], preferred_element_type=jnp.float32)
    blk = jnp.concatenate([w1_ref[...], u], axis=-1)
    v = jnp.kron(jnp.eye(8, dtype=jnp.float32), blk)
    prod = jnp.dot(e_ref[...], v, preferred_element_type=jnp.float32)
    o_ref[...] = prod.reshape(8 * BP, W2)

  return pl.pallas_call(
      body,
      grid=(E8 // BP,),
      in_specs=[
          pl.BlockSpec((BP, DE8), lambda i: (i, 0)),
          pl.BlockSpec((DE, P), lambda i: (0, 0)),
          pl.BlockSpec((P, OUT), lambda i: (0, 0)),
      ],
      out_specs=pl.BlockSpec((8 * BP, W2), lambda i: (i, 0)),
      out_shape=jax.ShapeDtypeStruct((8 * E8, W2), jnp.float32),
  )(edge_emb8, W_edge_prep, W_e_self_0)


def _sc_edge(F, EP, idx2):
  E2 = EP.shape[1]
  E = EP.shape[0]
  OUT = F.shape[1]
  EC = IDXC
  n_chunks = E // EC
  ipw = (n_chunks + NW - 1) // NW
  titers = (ipw + 2) // 3
  nj = OUT // LANES

  @functools.partial(
      pl.kernel,
      out_type=jax.ShapeDtypeStruct((E, E2), jnp.float32),
      mesh=_sc_mesh(),
      compiler_params=_SC_PARAMS,
      scratch_types=[
          pltpu.VMEM((ipw * 2 * EC,), jnp.int32),
          pltpu.VMEM((2 * EC, OUT), jnp.float32),
          pltpu.VMEM((2 * EC, OUT), jnp.float32),
          pltpu.VMEM((2 * EC, OUT), jnp.float32),
          pltpu.VMEM((EC, E2), jnp.float32),
          pltpu.VMEM((EC, E2), jnp.float32),
          pltpu.VMEM((EC, E2), jnp.float32),
          pltpu.SemaphoreType.DMA,
          pltpu.SemaphoreType.DMA,
          pltpu.SemaphoreType.DMA,
          pltpu.SemaphoreType.DMA,
          pltpu.SemaphoreType.DMA,
          pltpu.SemaphoreType.DMA,
      ],
  )
  def k(f_hbm, ep_hbm, idx_hbm, out_hbm,
        idx_all, sd0, sd1, sd2, ep0, ep1, ep2,
        si0, si1, si2, so0, so1, so2):
    wid = lax.axis_index("s") * NC + lax.axis_index("c")
    cbase = wid * ipw
    sd_b = (sd0, sd1, sd2)
    ep_b = (ep0, ep1, ep2)
    si_b = (si0, si1, si2)
    so_b = (so0, so1, so2)

    pltpu.sync_copy(idx_hbm.at[pl.ds(cbase * 2 * EC, ipw * 2 * EC)], idx_all)

    def cond(i):
      return (i < ipw) & (cbase + i < n_chunks)

    def fire(slot, i, first):
      @pl.when(cond(i))
      def _():
        if not first:
          @pl.when(i >= 3)
          def _():
            pltpu.make_async_copy(
                ep_b[slot], out_hbm.at[pl.ds(0, EC)], so_b[slot]).wait()
        off = i * 2 * EC
        pltpu.async_copy(
            f_hbm.at[idx_all.at[pl.ds(off, EC)]],
            sd_b[slot].at[pl.ds(0, EC)], si_b[slot])
        pltpu.async_copy(
            f_hbm.at[idx_all.at[pl.ds(off + EC, EC)]],
            sd_b[slot].at[pl.ds(EC, EC)], si_b[slot])
        pltpu.async_copy(
            ep_hbm.at[pl.ds((cbase + i) * EC, EC)], ep_b[slot], si_b[slot])

    def process(slot, i):
      @pl.when(cond(i))
      def _():
        pltpu.make_async_copy(
            f_hbm.at[idx_all.at[pl.ds(0, EC)]],
            sd_b[slot].at[pl.ds(0, EC)], si_b[slot]).wait()
        pltpu.make_async_copy(
            f_hbm.at[idx_all.at[pl.ds(0, EC)]],
            sd_b[slot].at[pl.ds(EC, EC)], si_b[slot]).wait()
        pltpu.make_async_copy(
            ep_hbm.at[pl.ds(0, EC)], ep_b[slot], si_b[slot]).wait()
        sd = sd_b[slot]
        ep = ep_b[slot]

        def inner(e, icarry):
          for j in range(nj):
            sl = pl.ds(j * LANES, LANES)
            sle = pl.ds(OUT + j * LANES, LANES)
            v = ep[e, sle] + sd[e, sl] + sd[EC + e, sl]
            ep[e, sle] = jnp.maximum(v, 0.0)
          return icarry

        lax.fori_loop(0, EC, inner, 0, unroll=2)
        pltpu.async_copy(
            ep, out_hbm.at[pl.ds((cbase + i) * EC, EC)], so_b[slot])

    fire(0, 0, True)
    fire(1, 1, True)

    def body(t, carry):
      i0 = 3 * t
      fire(2, i0 + 2, False)
      process(0, i0)
      fire(0, i0 + 3, False)
      process(1, i0 + 1)
      fire(1, i0 + 4, False)
      process(2, i0 + 2)
      return carry

    lax.fori_loop(0, titers, body, 0)

    for slot in range(3):
      @pl.when(cond(slot))
      def _():
        pltpu.make_async_copy(
            ep_b[slot], out_hbm.at[pl.ds(0, EC)], so_b[slot]).wait()

  return k(F, EP, idx2)


def _sc_nodesum(EB, n2e_flat, N, S):
  W = EB.shape[1]
  E = EB.shape[0]
  GC = 2 * IDXC // S
  n_chunks = N // GC
  ipw = (n_chunks + NW - 1) // NW
  titers = (ipw + 2) // 3
  nj = W // LANES

  @functools.partial(
      pl.kernel,
      out_type=jax.ShapeDtypeStruct((N, W), jnp.float32),
      mesh=_sc_mesh(),
      compiler_params=_SC_PARAMS,
      scratch_types=[
          pltpu.VMEM((ipw * 2 * IDXC,), jnp.int32),
          pltpu.VMEM((2 * IDXC, W), jnp.float32),
          pltpu.VMEM((2 * IDXC, W), jnp.float32),
          pltpu.VMEM((2 * IDXC, W), jnp.float32),
          pltpu.VMEM((GC, W), jnp.float32),
          pltpu.VMEM((GC, W), jnp.float32),
          pltpu.VMEM((GC, W), jnp.float32),
          pltpu.SemaphoreType.DMA,
          pltpu.SemaphoreType.DMA,
          pltpu.SemaphoreType.DMA,
          pltpu.SemaphoreType.DMA,
          pltpu.SemaphoreType.DMA,
          pltpu.SemaphoreType.DMA,
      ],
  )
  def k(eb_hbm, idx_hbm, out_hbm,
        idx_all, g0, g1, g2, o0, o1, o2, si0, si1, si2, so0, so1, so2):
    wid = lax.axis_index("s") * NC + lax.axis_index("c")
    cbase = wid * ipw
    g_b = (g0, g1, g2)
    o_b = (o0, o1, o2)
    si_b = (si0, si1, si2)
    so_b = (so0, so1, so2)

    pltpu.sync_copy(
        idx_hbm.at[pl.ds(cbase * 2 * IDXC, ipw * 2 * IDXC)], idx_all)

    def cond(i):
      return (i < ipw) & (cbase + i < n_chunks)

    def fire(slot, i):
      @pl.when(cond(i))
      def _():
        off = i * 2 * IDXC
        pltpu.async_copy(
            eb_hbm.at[idx_all.at[pl.ds(off, IDXC)]],
            g_b[slot].at[pl.ds(0, IDXC)], si_b[slot])
        pltpu.async_copy(
            eb_hbm.at[idx_all.at[pl.ds(off + IDXC, IDXC)]],
            g_b[slot].at[pl.ds(IDXC, IDXC)], si_b[slot])

    def process(slot, i, first):
      @pl.when(cond(i))
      def _():
        pltpu.make_async_copy(
            eb_hbm.at[idx_all.at[pl.ds(0, IDXC)]],
            g_b[slot].at[pl.ds(0, IDXC)], si_b[slot]).wait()
        pltpu.make_async_copy(
            eb_hbm.at[idx_all.at[pl.ds(0, IDXC)]],
            g_b[slot].at[pl.ds(IDXC, IDXC)], si_b[slot]).wait()
        if not first:
          @pl.when(i >= 3)
          def _():
            pltpu.make_async_copy(
                o_b[slot], out_hbm.at[pl.ds(0, GC)], so_b[slot]).wait()
        g = g_b[slot]
        o = o_b[slot]

        def node_body(kk, icarry):
          row = kk * S
          for j in range(nj):
            sl = pl.ds(j * LANES, LANES)
            acc = g[row, sl]
            for s in range(1, S):
              acc = acc + g[row + s, sl]
            o[kk, sl] = acc
          return icarry

        lax.fori_loop(0, GC, node_body, 0)
        pltpu.async_copy(
            o, out_hbm.at[pl.ds((cbase + i) * GC, GC)], so_b[slot])

    fire(0, 0)
    fire(1, 1)

    def body(t, carry):
      i0 = 3 * t
      fire(2, i0 + 2)
      process(0, i0, False)
      fire(0, i0 + 3)
      process(1, i0 + 1, False)
      fire(1, i0 + 4)
      process(2, i0 + 2, False)
      return carry

    lax.fori_loop(0, titers, body, 0)

    for slot in range(3):
      @pl.when(cond(slot))
      def _():
        pltpu.make_async_copy(
            o_b[slot], out_hbm.at[pl.ds(0, GC)], so_b[slot]).wait()

  return k(EB, n2e_flat)


def _tc_final(id_emb, sums, Wns0, Wnn0, Wns1, Wnn1, S):
  N, P = id_emb.shape
  OUT = Wns0.shape[1]
  BN = 2000
  inv_s = 1.0 / S

  def body(id_ref, s_ref, a0_ref, b0_ref, a1_ref, b1_ref, o_ref):
    m0 = s_ref[:, :OUT] * inv_s
    f1 = jnp.maximum(
        jnp.dot(id_ref[...], a0_ref[...], preferred_element_type=jnp.float32)
        + jnp.dot(m0, b0_ref[...], preferred_element_type=jnp.float32), 0.0)
    m1 = s_ref[:, OUT:] * inv_s
    f2 = jnp.maximum(
        jnp.dot(f1, a1_ref[...], preferred_element_type=jnp.float32)
        + jnp.dot(m1, b1_ref[...], preferred_element_type=jnp.float32), 0.0)
    o_ref[...] = jnp.concatenate([f1, f2], axis=-1)[None]

  return pl.pallas_call(
      body,
      grid=(N // BN,),
      in_specs=[
          pl.BlockSpec((BN, P), lambda i: (i, 0)),
          pl.BlockSpec((BN, 2 * OUT), lambda i: (i, 0)),
          pl.BlockSpec((P, OUT), lambda i: (0, 0)),
          pl.BlockSpec((OUT, OUT), lambda i: (0, 0)),
          pl.BlockSpec((OUT, OUT), lambda i: (0, 0)),
          pl.BlockSpec((OUT, OUT), lambda i: (0, 0)),
      ],
      out_specs=pl.BlockSpec((1, BN, 2 * OUT), lambda i: (0, i, 0)),
      out_shape=jax.ShapeDtypeStruct((1, N, 2 * OUT), jnp.float32),
  )(id_emb, sums, Wns0, Wnn0, Wns1, Wnn1)


def kernel(feats, node2edge_idx, edge_emb, edge_node_adj, id_emb,
           W_prep, W_edge_prep,
           W_e_self_0, W_e_neigh_0, W_n_self_0, W_n_neigh_0,
           W_e_self_1, W_e_neigh_1, W_n_self_1, W_n_neigh_1):
  N, S = node2edge_idx.shape
  E = edge_emb.shape[0]

  src = edge_node_adj[:, 0].reshape(-1, IDXC)
  dst = edge_node_adj[:, 1].reshape(-1, IDXC)
  idx2 = jnp.concatenate([src, dst], axis=1).reshape(-1)
  ipw_e = (E // IDXC + NW - 1) // NW
  idx2 = jnp.pad(idx2, (0, NW * ipw_e * 2 * IDXC - idx2.shape[0]))
  n2e_flat = node2edge_idx.reshape(-1)
  ipw_n = (N * S // (2 * IDXC) + NW - 1) // NW
  n2e_flat = jnp.pad(n2e_flat, (0, NW * ipw_n * 2 * IDXC - n2e_flat.shape[0]))

  F = _tc_node_prep(feats, W_prep, W_e_neigh_0)
  EP = _tc_edge_prep(edge_emb.reshape(E // 8, -1), W_edge_prep, W_e_self_0)
  EB = _sc_edge(F, EP, idx2)
  sums = _sc_nodesum(EB, n2e_flat, N, S)
  return _tc_final(id_emb, sums,
                   W_n_self_0, W_n_neigh_0, W_n_self_1, W_n_neigh_1, S)


# speedup vs baseline: 8.2925x; 1.0331x over previous
import functools

import jax
import jax.numpy as jnp
from jax import lax
from jax.experimental import pallas as pl
from jax.experimental.pallas import tpu as pltpu
from jax.experimental.pallas import tpu_sc as plsc

NC = 2
NS = 16
NW = NC * NS
LANES = 16
IDXC = 128


def _sc_mesh():
  return plsc.VectorSubcoreMesh(
      core_axis_name="c", subcore_axis_name="s", num_cores=NC, num_subcores=NS)


_SC_PARAMS = pltpu.CompilerParams(use_tc_tiling_on_sc=False)


def _tc_node_prep(feats, W_prep, W_e_neigh_0):
  N, DF = feats.shape
  P = W_prep.shape[1]
  OUT = W_e_neigh_0.shape[1]
  BN = 2000

  def body(f_ref, wp_ref, wn_ref, o_ref):
    f0 = jnp.dot(f_ref[...], wp_ref[...], preferred_element_type=jnp.float32)
    o_ref[...] = 0.5 * jnp.dot(f0, wn_ref[...],
                               preferred_element_type=jnp.float32)

  return pl.pallas_call(
      body,
      grid=(N // BN,),
      in_specs=[
          pl.BlockSpec((BN, DF), lambda i: (i, 0)),
          pl.BlockSpec((DF, P), lambda i: (0, 0)),
          pl.BlockSpec((P, OUT), lambda i: (0, 0)),
      ],
      out_specs=pl.BlockSpec((BN, OUT), lambda i: (i, 0)),
      out_shape=jax.ShapeDtypeStruct((N, OUT), jnp.float32),
  )(feats, W_prep, W_e_neigh_0)


def _tc_edge_prep(edge_emb8, W_edge_prep, W_e_self_0):
  E8, DE8 = edge_emb8.shape
  DE = DE8 // 8
  P = W_edge_prep.shape[1]
  OUT = W_e_self_0.shape[1]
  W2 = P + OUT
  BP = 1000

  def body(e_ref, w1_ref, w2_ref, o_ref):
    u = jnp.dot(w1_ref[...], w2_ref[...], preferred_element_type=jnp.float32)
    blk = jnp.concatenate([w1_ref[...], u], axis=-1)
    v = jnp.kron(jnp.eye(8, dtype=jnp.float32), blk)
    prod = jnp.dot(e_ref[...], v, preferred_element_type=jnp.float32)
    o_ref[...] = prod.reshape(8 * BP, W2)

  return pl.pallas_call(
      body,
      grid=(E8 // BP,),
      in_specs=[
          pl.BlockSpec((BP, DE8), lambda i: (i, 0)),
          pl.BlockSpec((DE, P), lambda i: (0, 0)),
          pl.BlockSpec((P, OUT), lambda i: (0, 0)),
      ],
      out_specs=pl.BlockSpec((8 * BP, W2), lambda i: (i, 0)),
      out_shape=jax.ShapeDtypeStruct((8 * E8, W2), jnp.float32),
  )(edge_emb8, W_edge_prep, W_e_self_0)


def _sc_edge(F, EP, idx2):
  E2 = EP.shape[1]
  E = EP.shape[0]
  OUT = F.shape[1]
  EC = IDXC
  n_chunks = E // EC
  ipw = (n_chunks + NW - 1) // NW
  titers = (ipw + 2) // 3
  nj = OUT // LANES

  @functools.partial(
      pl.kernel,
      out_type=jax.ShapeDtypeStruct((E, OUT), jnp.float32),
      mesh=_sc_mesh(),
      compiler_params=_SC_PARAMS,
      scratch_types=[
          pltpu.VMEM((ipw * 2 * EC,), jnp.int32),
          pltpu.VMEM((2 * EC, OUT), jnp.float32),
          pltpu.VMEM((2 * EC, OUT), jnp.float32),
          pltpu.VMEM((2 * EC, OUT), jnp.float32),
          pltpu.VMEM((EC, OUT), jnp.float32),
          pltpu.VMEM((EC, OUT), jnp.float32),
          pltpu.VMEM((EC, OUT), jnp.float32),
          pltpu.SemaphoreType.DMA,
          pltpu.SemaphoreType.DMA,
          pltpu.SemaphoreType.DMA,
          pltpu.SemaphoreType.DMA,
          pltpu.SemaphoreType.DMA,
          pltpu.SemaphoreType.DMA,
      ],
  )
  def k(f_hbm, ep_hbm, idx_hbm, out_hbm,
        idx_all, sd0, sd1, sd2, ep0, ep1, ep2,
        si0, si1, si2, so0, so1, so2):
    wid = lax.axis_index("s") * NC + lax.axis_index("c")
    cbase = wid * ipw
    sd_b = (sd0, sd1, sd2)
    ep_b = (ep0, ep1, ep2)
    si_b = (si0, si1, si2)
    so_b = (so0, so1, so2)

    pltpu.sync_copy(idx_hbm.at[pl.ds(cbase * 2 * EC, ipw * 2 * EC)], idx_all)

    def cond(i):
      return (i < ipw) & (cbase + i < n_chunks)

    def fire(slot, i, first):
      @pl.when(cond(i))
      def _():
        if not first:
          @pl.when(i >= 3)
          def _():
            pltpu.make_async_copy(
                ep_b[slot], out_hbm.at[pl.ds(0, EC)], so_b[slot]).wait()
        off = i * 2 * EC
        pltpu.async_copy(
            f_hbm.at[idx_all.at[pl.ds(off, EC)]],
            sd_b[slot].at[pl.ds(0, EC)], si_b[slot])
        pltpu.async_copy(
            f_hbm.at[idx_all.at[pl.ds(off + EC, EC)]],
            sd_b[slot].at[pl.ds(EC, EC)], si_b[slot])
        pltpu.async_copy(
            ep_hbm.at[pl.ds((cbase + i) * EC, EC), pl.ds(OUT, OUT)],
            ep_b[slot], si_b[slot])

    def process(slot, i):
      @pl.when(cond(i))
      def _():
        pltpu.make_async_copy(
            f_hbm.at[idx_all.at[pl.ds(0, EC)]],
            sd_b[slot].at[pl.ds(0, EC)], si_b[slot]).wait()
        pltpu.make_async_copy(
            f_hbm.at[idx_all.at[pl.ds(0, EC)]],
            sd_b[slot].at[pl.ds(EC, EC)], si_b[slot]).wait()
        pltpu.make_async_copy(
            ep_hbm.at[pl.ds(0, EC), pl.ds(OUT, OUT)],
            ep_b[slot], si_b[slot]).wait()
        sd = sd_b[slot]
        ep = ep_b[slot]

        def inner(e, icarry):
          for j in range(nj):
            sl = pl.ds(j * LANES, LANES)
            v = ep[e, sl] + sd[e, sl] + sd[EC + e, sl]
            ep[e, sl] = jnp.maximum(v, 0.0)
          return icarry

        lax.fori_loop(0, EC, inner, 0, unroll=2)
        pltpu.async_copy(
            ep, out_hbm.at[pl.ds((cbase + i) * EC, EC)], so_b[slot])

    fire(0, 0, True)
    fire(1, 1, True)

    def body(t, carry):
      i0 = 3 * t
      fire(2, i0 + 2, False)
      process(0, i0)
      fire(0, i0 + 3, False)
      process(1, i0 + 1)
      fire(1, i0 + 4, False)
      process(2, i0 + 2)
      return carry

    lax.fori_loop(0, titers, body, 0)

    for slot in range(3):
      @pl.when(cond(slot))
      def _():
        pltpu.make_async_copy(
            ep_b[slot], out_hbm.at[pl.ds(0, EC)], so_b[slot]).wait()

  return k(F, EP, idx2)


def _sc_nodesum(EP, EB1, n2e_flat, N, S):
  W2 = EP.shape[1]
  OUT = EB1.shape[1]
  GC = 2 * IDXC // S
  n_chunks = N // GC
  ipw = (n_chunks + NW - 1) // NW
  titers = (ipw + 1) // 2
  nj = OUT // LANES

  @functools.partial(
      pl.kernel,
      out_type=jax.ShapeDtypeStruct((N, 2 * OUT), jnp.float32),
      mesh=_sc_mesh(),
      compiler_params=_SC_PARAMS,
      scratch_types=[
          pltpu.VMEM((ipw * 2 * IDXC,), jnp.int32),
          pltpu.VMEM((2 * IDXC, W2), jnp.float32),
          pltpu.VMEM((2 * IDXC, W2), jnp.float32),
          pltpu.VMEM((2 * IDXC, OUT), jnp.float32),
          pltpu.VMEM((2 * IDXC, OUT), jnp.float32),
          pltpu.VMEM((GC, 2 * OUT), jnp.float32),
          pltpu.VMEM((GC, 2 * OUT), jnp.float32),
          pltpu.SemaphoreType.DMA,
          pltpu.SemaphoreType.DMA,
          pltpu.SemaphoreType.DMA,
          pltpu.SemaphoreType.DMA,
      ],
  )
  def k(ep_hbm, eb_hbm, idx_hbm, out_hbm,
        idx_all, gp0, gp1, gb0, gb1, o0, o1, si0, si1, so0, so1):
    wid = lax.axis_index("s") * NC + lax.axis_index("c")
    cbase = wid * ipw
    gp_b = (gp0, gp1)
    gb_b = (gb0, gb1)
    o_b = (o0, o1)
    si_b = (si0, si1)
    so_b = (so0, so1)

    pltpu.sync_copy(
        idx_hbm.at[pl.ds(cbase * 2 * IDXC, ipw * 2 * IDXC)], idx_all)

    def cond(i):
      return (i < ipw) & (cbase + i < n_chunks)

    def fire(slot, i):
      @pl.when(cond(i))
      def _():
        off = i * 2 * IDXC
        for h in range(2):
          pltpu.async_copy(
              ep_hbm.at[idx_all.at[pl.ds(off + h * IDXC, IDXC)]],
              gp_b[slot].at[pl.ds(h * IDXC, IDXC)], si_b[slot])
          pltpu.async_copy(
              eb_hbm.at[idx_all.at[pl.ds(off + h * IDXC, IDXC)]],
              gb_b[slot].at[pl.ds(h * IDXC, IDXC)], si_b[slot])

    def process(slot, i):
      @pl.when(cond(i))
      def _():
        for h in range(2):
          pltpu.make_async_copy(
              ep_hbm.at[idx_all.at[pl.ds(0, IDXC)]],
              gp_b[slot].at[pl.ds(h * IDXC, IDXC)], si_b[slot]).wait()
          pltpu.make_async_copy(
              eb_hbm.at[idx_all.at[pl.ds(0, IDXC)]],
              gb_b[slot].at[pl.ds(h * IDXC, IDXC)], si_b[slot]).wait()

        @pl.when(i >= 2)
        def _():
          pltpu.make_async_copy(
              o_b[slot], out_hbm.at[pl.ds(0, GC)], so_b[slot]).wait()
        gp = gp_b[slot]
        gb = gb_b[slot]
        o = o_b[slot]

        def node_body(kk, icarry):
          row = kk * S
          for j in range(nj):
            sl = pl.ds(j * LANES, LANES)
            acc = gp[row, sl]
            for s in range(1, S):
              acc = acc + gp[row + s, sl]
            o[kk, sl] = acc
            acc1 = gb[row, sl]
            for s in range(1, S):
              acc1 = acc1 + gb[row + s, sl]
            o[kk, pl.ds(OUT + j * LANES, LANES)] = acc1
          return icarry

        lax.fori_loop(0, GC, node_body, 0)
        pltpu.async_copy(
            o, out_hbm.at[pl.ds((cbase + i) * GC, GC)], so_b[slot])

    fire(0, 0)
    fire(1, 1)

    def body(t, carry):
      i0 = 2 * t
      process(0, i0)
      fire(0, i0 + 2)
      process(1, i0 + 1)
      fire(1, i0 + 3)
      return carry

    lax.fori_loop(0, titers, body, 0)

    for slot in range(2):
      @pl.when(cond(slot))
      def _():
        pltpu.make_async_copy(
            o_b[slot], out_hbm.at[pl.ds(0, GC)], so_b[slot]).wait()

  return k(EP, EB1, n2e_flat)


def _tc_final(id_emb, sums, Wns0, Wnn0, Wns1, Wnn1, S):
  N, P = id_emb.shape
  OUT = Wns0.shape[1]
  BN = 2000
  inv_s = 1.0 / S

  def body(id_ref, s_ref, a0_ref, b0_ref, a1_ref, b1_ref, o_ref):
    m0 = s_ref[:, :OUT] * inv_s
    f1 = jnp.maximum(
        jnp.dot(id_ref[...], a0_ref[...], preferred_element_type=jnp.float32)
        + jnp.dot(m0, b0_ref[...], preferred_element_type=jnp.float32), 0.0)
    m1 = s_ref[:, OUT:] * inv_s
    f2 = jnp.maximum(
        jnp.dot(f1, a1_ref[...], preferred_element_type=jnp.float32)
        + jnp.dot(m1, b1_ref[...], preferred_element_type=jnp.float32), 0.0)
    o_ref[...] = jnp.concatenate([f1, f2], axis=-1)[None]

  return pl.pallas_call(
      body,
      grid=(N // BN,),
      in_specs=[
          pl.BlockSpec((BN, P), lambda i: (i, 0)),
          pl.BlockSpec((BN, 2 * OUT), lambda i: (i, 0)),
          pl.BlockSpec((P, OUT), lambda i: (0, 0)),
          pl.BlockSpec((OUT, OUT), lambda i: (0, 0)),
          pl.BlockSpec((OUT, OUT), lambda i: (0, 0)),
          pl.BlockSpec((OUT, OUT), lambda i: (0, 0)),
      ],
      out_specs=pl.BlockSpec((1, BN, 2 * OUT), lambda i: (0, i, 0)),
      out_shape=jax.ShapeDtypeStruct((1, N, 2 * OUT), jnp.float32),
  )(id_emb, sums, Wns0, Wnn0, Wns1, Wnn1)


def kernel(feats, node2edge_idx, edge_emb, edge_node_adj, id_emb,
           W_prep, W_edge_prep,
           W_e_self_0, W_e_neigh_0, W_n_self_0, W_n_neigh_0,
           W_e_self_1, W_e_neigh_1, W_n_self_1, W_n_neigh_1):
  N, S = node2edge_idx.shape
  E = edge_emb.shape[0]

  src = edge_node_adj[:, 0].reshape(-1, IDXC)
  dst = edge_node_adj[:, 1].reshape(-1, IDXC)
  idx2 = jnp.concatenate([src, dst], axis=1).reshape(-1)
  ipw_e = (E // IDXC + NW - 1) // NW
  idx2 = jnp.pad(idx2, (0, NW * ipw_e * 2 * IDXC - idx2.shape[0]))
  n2e_flat = node2edge_idx.reshape(-1)
  ipw_n = (N * S // (2 * IDXC) + NW - 1) // NW
  n2e_flat = jnp.pad(n2e_flat, (0, NW * ipw_n * 2 * IDXC - n2e_flat.shape[0]))

  F = _tc_node_prep(feats, W_prep, W_e_neigh_0)
  EP = _tc_edge_prep(edge_emb.reshape(E // 8, -1), W_edge_prep, W_e_self_0)
  EB = _sc_edge(F, EP, idx2)
  sums = _sc_nodesum(EP, EB, n2e_flat, N, S)
  return _tc_final(id_emb, sums,
                   W_n_self_0, W_n_neigh_0, W_n_self_1, W_n_neigh_1, S)
